# Initial kernel scaffold; baseline (speedup 1.0000x reference)
import functools

import jax
import jax.numpy as jnp
from jax import lax
from jax.experimental import pallas as pl
from jax.experimental.pallas import tpu as pltpu
from jax.experimental.pallas import tpu_sc as plsc

_N = 10000
_E = 640000
_NP = 10240
_FLAT = 3 * _NP
_NC = 2
_NS = 16
_L = 16
_NW = _NC * _NS
_EPW = _E // _NW
_STEPS = _EPW // _L
_RED = _FLAT // _NS


def _fast_rsqrt(d2):
    half = jnp.float32(0.5)
    three_half = jnp.float32(1.5)
    ibits = plsc.bitcast(d2, jnp.int32)
    y = plsc.bitcast(jnp.int32(0x5F3759DF) - lax.shift_right_logical(ibits, 1),
                     jnp.float32)
    hd2 = half * d2
    y = y * (three_half - hd2 * y * y)
    y = y * (three_half - hd2 * y * y)
    y = y * (three_half - hd2 * y * y)
    return y


def _sc_body(pos_hbm, edge_hbm, fparts_hbm, eparts_hbm,
             pos_v, src_v, dst_v, facc, e_v, tmp_v, acc_v, shared):
    cid = lax.axis_index("c")
    sid = lax.axis_index("s")
    wid = sid * _NC + cid
    base = wid * _EPW

    pltpu.sync_copy(pos_hbm, pos_v)
    pltpu.sync_copy(edge_hbm.at[0, pl.ds(base, _EPW)], src_v)
    pltpu.sync_copy(edge_hbm.at[1, pl.ds(base, _EPW)], dst_v)

    zeros = jnp.zeros((_L,), jnp.float32)

    def zero_body(k, _):
        facc[pl.ds(k * _L, _L)] = zeros
        return 0

    lax.fori_loop(0, _FLAT // _L, zero_body, 0)

    one = jnp.float32(1.0)
    half = jnp.float32(0.5)
    fzero = jnp.float32(0.0)

    def edge_body(k, eacc):
        s = src_v[pl.ds(k * _L, _L)]
        t = dst_v[pl.ds(k * _L, _L)]
        xi = plsc.load_gather(pos_v, [s])
        yi = plsc.load_gather(pos_v, [s + _NP])
        zi = plsc.load_gather(pos_v, [s + 2 * _NP])
        xj = plsc.load_gather(pos_v, [t])
        yj = plsc.load_gather(pos_v, [t + _NP])
        zj = plsc.load_gather(pos_v, [t + 2 * _NP])
        dx = xj - xi
        dy = yj - yi
        dz = zj - zi
        d2 = dx * dx + dy * dy + dz * dz
        r = _fast_rsqrt(d2)
        d = d2 * r
        mask = d2 < one
        omd = one - d
        e = jnp.where(mask, half * omd * omd, fzero)
        g = jnp.where(mask, one - r, fzero)
        g = jnp.where(d2 == fzero, fzero, g)
        fx = g * dx
        fy = g * dy
        fz = g * dz
        plsc.addupdate_scatter(facc, [s], fx)
        plsc.addupdate_scatter(facc, [s + _NP], fy)
        plsc.addupdate_scatter(facc, [s + 2 * _NP], fz)
        plsc.addupdate_scatter(facc, [t], -fx)
        plsc.addupdate_scatter(facc, [t + _NP], -fy)
        plsc.addupdate_scatter(facc, [t + 2 * _NP], -fz)
        return eacc + e

    eacc = lax.fori_loop(0, _STEPS, edge_body, zeros)

    e_v[...] = eacc
    pltpu.sync_copy(e_v, eparts_hbm.at[wid])

    pltpu.sync_copy(facc, shared.at[sid])
    plsc.subcore_barrier()

    off = sid * _RED
    pltpu.sync_copy(shared.at[0, pl.ds(off, _RED)], acc_v)
    for p in range(1, _NS):
        pltpu.sync_copy(shared.at[p, pl.ds(off, _RED)], tmp_v)

        def add_body(k, _):
            acc_v[pl.ds(k * _L, _L)] = (acc_v[pl.ds(k * _L, _L)]
                                        + tmp_v[pl.ds(k * _L, _L)])
            return 0

        lax.fori_loop(0, _RED // _L, add_body, 0)
    pltpu.sync_copy(acc_v, fparts_hbm.at[cid, pl.ds(off, _RED)])


_sc_call = pl.kernel(
    _sc_body,
    out_type=(
        jax.ShapeDtypeStruct((_NC, _FLAT), jnp.float32),
        jax.ShapeDtypeStruct((_NW, _L), jnp.float32),
    ),
    mesh=plsc.VectorSubcoreMesh(
        core_axis_name="c", subcore_axis_name="s",
        num_cores=_NC, num_subcores=_NS),
    scratch_types=[
        pltpu.VMEM((_FLAT,), jnp.float32),
        pltpu.VMEM((_EPW,), jnp.int32),
        pltpu.VMEM((_EPW,), jnp.int32),
        pltpu.VMEM((_FLAT,), jnp.float32),
        pltpu.VMEM((_L,), jnp.float32),
        pltpu.VMEM((_RED,), jnp.float32),
        pltpu.VMEM((_RED,), jnp.float32),
        pltpu.VMEM_SHARED((_NS, _FLAT), jnp.float32),
    ],
)


def _tc_body(fp_ref, ep_ref, f_ref, e_ref):
    f_ref[...] = fp_ref[0] + fp_ref[1]
    e_ref[0, 0] = jnp.float32(0.5) * jnp.sum(ep_ref[...])


_tc_call = pl.pallas_call(
    _tc_body,
    out_shape=(
        jax.ShapeDtypeStruct((_FLAT // 128, 128), jnp.float32),
        jax.ShapeDtypeStruct((1, 1), jnp.float32),
    ),
    in_specs=[
        pl.BlockSpec(memory_space=pltpu.VMEM),
        pl.BlockSpec(memory_space=pltpu.VMEM),
    ],
    out_specs=(
        pl.BlockSpec(memory_space=pltpu.VMEM),
        pl.BlockSpec(memory_space=pltpu.SMEM),
    ),
)


@jax.jit
def kernel(positions, edge_index):
    pos_flat = (jnp.zeros((3, _NP), jnp.float32)
                .at[:, :_N].set(positions.T).reshape(_FLAT))
    fparts, eparts = _sc_call(pos_flat, edge_index)
    f2d, e = _tc_call(fparts.reshape(_NC, _FLAT // 128, 128), eparts)
    flat = f2d.reshape(_FLAT)
    forces = jnp.stack(
        [flat[:_N], flat[_NP:_NP + _N], flat[2 * _NP:2 * _NP + _N]], axis=1)
    return e[0, 0], forces


# speedup vs baseline: 69.5182x; 69.5182x over previous
import functools

import jax
import jax.numpy as jnp
from jax import lax
from jax.experimental import pallas as pl
from jax.experimental.pallas import tpu as pltpu
from jax.experimental.pallas import tpu_sc as plsc

_N = 10000
_E = 640000
_NP = 10240
_FLAT = 3 * _NP
_NC = 2
_NS = 16
_L = 16
_NW = _NC * _NS
_EPW = _E // _NW
_STEPS = _EPW // _L
_NROUND = 3
_SLAB = _FLAT // _NROUND
_RED = _SLAB // _NS


def _fast_rsqrt(d2):
    half = jnp.float32(0.5)
    three_half = jnp.float32(1.5)
    ibits = plsc.bitcast(d2, jnp.int32)
    y = plsc.bitcast(jnp.int32(0x5F3759DF) - lax.shift_right_logical(ibits, 1),
                     jnp.float32)
    hd2 = half * d2
    y = y * (three_half - hd2 * y * y)
    y = y * (three_half - hd2 * y * y)
    y = y * (three_half - hd2 * y * y)
    return y


def _sc_body(pos_hbm, src_hbm, dst_hbm, fparts_hbm, eparts_hbm,
             pos_v, src_v, dst_v, facc, e_v, tmp_v, acc_v, shared):
    cid = lax.axis_index("c")
    sid = lax.axis_index("s")
    wid = sid * _NC + cid
    base = wid * _EPW

    pltpu.sync_copy(pos_hbm, pos_v)
    pltpu.sync_copy(src_hbm.at[pl.ds(base, _EPW)], src_v)
    pltpu.sync_copy(dst_hbm.at[pl.ds(base, _EPW)], dst_v)

    zeros = jnp.zeros((_L,), jnp.float32)

    def zero_body(k, _):
        facc[pl.ds(k * _L, _L)] = zeros
        return 0

    lax.fori_loop(0, _FLAT // _L, zero_body, 0)

    one = jnp.float32(1.0)
    half = jnp.float32(0.5)
    fzero = jnp.float32(0.0)

    def edge_body(k, eacc):
        s = src_v[pl.ds(k * _L, _L)]
        t = dst_v[pl.ds(k * _L, _L)]
        xi = plsc.load_gather(pos_v, [s])
        yi = plsc.load_gather(pos_v, [s + _NP])
        zi = plsc.load_gather(pos_v, [s + 2 * _NP])
        xj = plsc.load_gather(pos_v, [t])
        yj = plsc.load_gather(pos_v, [t + _NP])
        zj = plsc.load_gather(pos_v, [t + 2 * _NP])
        dx = xj - xi
        dy = yj - yi
        dz = zj - zi
        d2 = dx * dx + dy * dy + dz * dz
        r = _fast_rsqrt(d2)
        d = d2 * r
        mask = d2 < one
        omd = one - d
        e = jnp.where(mask, half * omd * omd, fzero)
        g = jnp.where(mask, one - r, fzero)
        g = jnp.where(d2 == fzero, fzero, g)
        fx = g * dx
        fy = g * dy
        fz = g * dz
        plsc.addupdate_scatter(facc, [s], fx)
        plsc.addupdate_scatter(facc, [s + _NP], fy)
        plsc.addupdate_scatter(facc, [s + 2 * _NP], fz)
        plsc.addupdate_scatter(facc, [t], -fx)
        plsc.addupdate_scatter(facc, [t + _NP], -fy)
        plsc.addupdate_scatter(facc, [t + 2 * _NP], -fz)
        return eacc + e

    eacc = lax.fori_loop(0, _STEPS, edge_body, zeros)

    e_v[...] = eacc
    pltpu.sync_copy(e_v, eparts_hbm.at[pl.ds(wid * _L, _L)])

    for rnd in range(_NROUND):
        pltpu.sync_copy(facc.at[pl.ds(rnd * _SLAB, _SLAB)],
                        shared.at[pl.ds(sid * _SLAB, _SLAB)])
        plsc.subcore_barrier()

        off = sid * _RED
        pltpu.sync_copy(shared.at[pl.ds(off, _RED)], acc_v)
        for p in range(1, _NS):
            pltpu.sync_copy(shared.at[pl.ds(p * _SLAB + off, _RED)], tmp_v)

            def add_body(k, _):
                acc_v[pl.ds(k * _L, _L)] = (acc_v[pl.ds(k * _L, _L)]
                                            + tmp_v[pl.ds(k * _L, _L)])
                return 0

            lax.fori_loop(0, _RED // _L, add_body, 0)
        pltpu.sync_copy(
            acc_v,
            fparts_hbm.at[pl.ds(cid * _FLAT + rnd * _SLAB + off, _RED)])
        plsc.subcore_barrier()


_sc_call = pl.kernel(
    _sc_body,
    out_type=(
        jax.ShapeDtypeStruct((_NC * _FLAT,), jnp.float32),
        jax.ShapeDtypeStruct((_NW * _L,), jnp.float32),
    ),
    mesh=plsc.VectorSubcoreMesh(
        core_axis_name="c", subcore_axis_name="s",
        num_cores=_NC, num_subcores=_NS),
    scratch_types=[
        pltpu.VMEM((_FLAT,), jnp.float32),
        pltpu.VMEM((_EPW,), jnp.int32),
        pltpu.VMEM((_EPW,), jnp.int32),
        pltpu.VMEM((_FLAT,), jnp.float32),
        pltpu.VMEM((_L,), jnp.float32),
        pltpu.VMEM((_RED,), jnp.float32),
        pltpu.VMEM((_RED,), jnp.float32),
        pltpu.VMEM_SHARED((_NS * _SLAB,), jnp.float32),
    ],
    compiler_params=pltpu.CompilerParams(needs_layout_passes=False),
)


def _tc_body(fp_ref, ep_ref, f_ref, e_ref):
    f_ref[...] = fp_ref[0] + fp_ref[1]
    e_ref[0, 0] = jnp.float32(0.5) * jnp.sum(ep_ref[...])


_ROWS = _FLAT // 128


_tc_call = pl.pallas_call(
    _tc_body,
    out_shape=(
        jax.ShapeDtypeStruct((_FLAT // 128, 128), jnp.float32),
        jax.ShapeDtypeStruct((1, 1), jnp.float32),
    ),
    grid=(),
    in_specs=[
        pl.BlockSpec(memory_space=pltpu.VMEM),
        pl.BlockSpec(memory_space=pltpu.VMEM),
    ],
    out_specs=(
        pl.BlockSpec(memory_space=pltpu.VMEM),
        pl.BlockSpec(memory_space=pltpu.SMEM),
    ),
)


@jax.jit
def kernel(positions, edge_index):
    pos_flat = (jnp.zeros((3, _NP), jnp.float32)
                .at[:, :_N].set(positions.T).reshape(_FLAT))
    fparts, eparts = _sc_call(pos_flat, edge_index[0], edge_index[1])
    f2d, e = _tc_call(fparts.reshape(_NC, _FLAT // 128, 128),
                      eparts.reshape(4, 128))
    flat = f2d.reshape(_FLAT)
    forces = jnp.stack(
        [flat[:_N], flat[_NP:_NP + _N], flat[2 * _NP:2 * _NP + _N]], axis=1)
    return e[0, 0], forces


# speedup vs baseline: 97.8263x; 1.4072x over previous
import jax
import jax.numpy as jnp
from jax import lax
from jax.experimental import pallas as pl
from jax.experimental.pallas import tpu as pltpu
from jax.experimental.pallas import tpu_sc as plsc

_N = 10000
_E = 640000
_W = 3 * _N
_FLAT = 30720
_NC = 2
_NS = 16
_L = 16
_NW = _NC * _NS
_EPW = _E // _NW
_STEPS = _EPW // _L
_NROUND = 3
_SLAB = _FLAT // _NROUND
_RED = _SLAB // _NS


def _fast_rsqrt(d2):
    half = jnp.float32(0.5)
    three_half = jnp.float32(1.5)
    ibits = plsc.bitcast(d2, jnp.int32)
    y = plsc.bitcast(jnp.int32(0x5F3759DF) - lax.shift_right_logical(ibits, 1),
                     jnp.float32)
    hd2 = half * d2
    y = y * (three_half - hd2 * y * y)
    y = y * (three_half - hd2 * y * y)
    y = y * (three_half - hd2 * y * y)
    return y


def _sc_body(pos_hbm, src_hbm, dst_hbm, fparts_hbm, eparts_hbm,
             pos_v, src_v, dst_v, facc, e_v, tmp_v, acc_v, shared,
             sem_in, sem_red):
    cid = lax.axis_index("c")
    sid = lax.axis_index("s")
    wid = sid * _NC + cid
    base = wid * _EPW

    cp_pos = pltpu.async_copy(pos_hbm, pos_v, sem_in)
    cp_src = pltpu.async_copy(src_hbm.at[pl.ds(base, _EPW)], src_v, sem_in)
    cp_dst = pltpu.async_copy(dst_hbm.at[pl.ds(base, _EPW)], dst_v, sem_in)

    zeros = jnp.zeros((_L,), jnp.float32)

    def zero_body(k, _):
        for j in range(16):
            facc[pl.ds(k * 256 + j * _L, _L)] = zeros
        return 0

    lax.fori_loop(0, _FLAT // 256, zero_body, 0)

    cp_pos.wait()
    cp_src.wait()
    cp_dst.wait()

    one = jnp.float32(1.0)
    half = jnp.float32(0.5)
    fzero = jnp.float32(0.0)

    @plsc.parallel_loop(0, _STEPS, 1, unroll=4,
                        carry=jnp.zeros((_L,), jnp.float32))
    def eacc(k, acc):
        s = src_v[pl.ds(k * _L, _L)]
        t = dst_v[pl.ds(k * _L, _L)]
        s3 = s + s + s
        t3 = t + t + t
        xi = plsc.load_gather(pos_v, [s3])
        yi = plsc.load_gather(pos_v, [s3 + 1])
        zi = plsc.load_gather(pos_v, [s3 + 2])
        xj = plsc.load_gather(pos_v, [t3])
        yj = plsc.load_gather(pos_v, [t3 + 1])
        zj = plsc.load_gather(pos_v, [t3 + 2])
        dx = xj - xi
        dy = yj - yi
        dz = zj - zi
        d2 = dx * dx + dy * dy + dz * dz
        r = _fast_rsqrt(d2)
        d = d2 * r
        mask = d2 < one
        omd = one - d
        e = jnp.where(mask, half * omd * omd, fzero)
        g = jnp.where(mask, one - r, fzero)
        g = jnp.where(d2 == fzero, fzero, g)
        fx = g * dx
        fy = g * dy
        fz = g * dz
        plsc.addupdate_scatter(facc, [s3], fx)
        plsc.addupdate_scatter(facc, [s3 + 1], fy)
        plsc.addupdate_scatter(facc, [s3 + 2], fz)
        plsc.addupdate_scatter(facc, [t3], -fx)
        plsc.addupdate_scatter(facc, [t3 + 1], -fy)
        plsc.addupdate_scatter(facc, [t3 + 2], -fz)
        return acc + e

    e_v[...] = eacc
    pltpu.sync_copy(e_v, eparts_hbm.at[pl.ds(wid * _L, _L)])

    for rnd in range(_NROUND):
        pltpu.sync_copy(facc.at[pl.ds(rnd * _SLAB, _SLAB)],
                        shared.at[pl.ds(sid * _SLAB, _SLAB)])
        plsc.subcore_barrier()

        off = sid * _RED
        cps = [
            pltpu.async_copy(
                shared.at[pl.ds(p * _SLAB + off, _RED)],
                tmp_v.at[pl.ds(p * _RED, _RED)], sem_red)
            for p in range(1, _NS)
        ]
        pltpu.sync_copy(shared.at[pl.ds(off, _RED)], acc_v)
        for cp in cps:
            cp.wait()

        def add_body(k, _):
            acc = acc_v[pl.ds(k * _L, _L)]
            for p in range(1, _NS):
                acc = acc + tmp_v[pl.ds(p * _RED + k * _L, _L)]
            acc_v[pl.ds(k * _L, _L)] = acc
            return 0

        lax.fori_loop(0, _RED // _L, add_body, 0)
        pltpu.sync_copy(
            acc_v,
            fparts_hbm.at[pl.ds(cid * _FLAT + rnd * _SLAB + off, _RED)])
        plsc.subcore_barrier()


_sc_call = pl.kernel(
    _sc_body,
    out_type=(
        jax.ShapeDtypeStruct((_NC * _FLAT,), jnp.float32),
        jax.ShapeDtypeStruct((_NW * _L,), jnp.float32),
    ),
    mesh=plsc.VectorSubcoreMesh(
        core_axis_name="c", subcore_axis_name="s",
        num_cores=_NC, num_subcores=_NS),
    scratch_types=[
        pltpu.VMEM((_W,), jnp.float32),
        pltpu.VMEM((_EPW,), jnp.int32),
        pltpu.VMEM((_EPW,), jnp.int32),
        pltpu.VMEM((_FLAT,), jnp.float32),
        pltpu.VMEM((_L,), jnp.float32),
        pltpu.VMEM((_NS * _RED,), jnp.float32),
        pltpu.VMEM((_RED,), jnp.float32),
        pltpu.VMEM_SHARED((_NS * _SLAB,), jnp.float32),
        pltpu.SemaphoreType.DMA,
        pltpu.SemaphoreType.DMA,
    ],
    compiler_params=pltpu.CompilerParams(needs_layout_passes=False),
)


def _tc_body(fp_ref, ep_ref, f_ref, e_ref):
    f_ref[...] = fp_ref[0] + fp_ref[1]
    e_ref[0, 0] = jnp.float32(0.5) * jnp.sum(ep_ref[...])


_tc_call = pl.pallas_call(
    _tc_body,
    out_shape=(
        jax.ShapeDtypeStruct((_FLAT // 128, 128), jnp.float32),
        jax.ShapeDtypeStruct((1, 1), jnp.float32),
    ),
    grid=(),
    in_specs=[
        pl.BlockSpec(memory_space=pltpu.VMEM),
        pl.BlockSpec(memory_space=pltpu.VMEM),
    ],
    out_specs=(
        pl.BlockSpec(memory_space=pltpu.VMEM),
        pl.BlockSpec(memory_space=pltpu.SMEM),
    ),
)


@jax.jit
def kernel(positions, edge_index):
    pos_flat = positions.reshape(_W)
    fparts, eparts = _sc_call(pos_flat, edge_index[0], edge_index[1])
    f2d, e = _tc_call(fparts.reshape(_NC, _FLAT // 128, 128),
                      eparts.reshape(4, 128))
    forces = f2d.reshape(_FLAT)[:_W].reshape(_N, 3)
    return e[0, 0], forces


# speedup vs baseline: 106.1243x; 1.0848x over previous
import jax
import jax.numpy as jnp
from jax import lax
from jax.experimental import pallas as pl
from jax.experimental.pallas import tpu as pltpu
from jax.experimental.pallas import tpu_sc as plsc

_N = 10000
_E = 640000
_W = 3 * _N
_FLAT = 30720
_NC = 2
_NS = 16
_L = 16
_NW = _NC * _NS
_COLS = _E // 128
_CA = _COLS // _NW
_NB = _COLS - _CA * _NW
_NA = _NW - _NB
_STAGE = (_CA + 1) * 128
_STEPS = _STAGE // _L
_NROUND = 3
_SLAB = _FLAT // _NROUND
_RED = _SLAB // _NS


def _fast_rsqrt(d2):
    half = jnp.float32(0.5)
    three_half = jnp.float32(1.5)
    ibits = plsc.bitcast(d2, jnp.int32)
    y = plsc.bitcast(jnp.int32(0x5F3759DF) - lax.shift_right_logical(ibits, 1),
                     jnp.float32)
    hd2 = half * d2
    y = y * (three_half - hd2 * y * y)
    y = y * (three_half - hd2 * y * y)
    y = y * (three_half - hd2 * y * y)
    return y


def _sc_body(pos_hbm, edge_hbm, fparts_hbm, eparts_hbm,
             pos_v, ev, facc, e_v, tmp_v, acc_v, shared,
             sem_in, sem_red):
    cid = lax.axis_index("c")
    sid = lax.axis_index("s")
    wid = sid * _NC + cid
    col_base = jnp.where(wid < _NA, wid * _CA, _NA * _CA + (wid - _NA) * (_CA + 1))
    cnt = jnp.where(wid < _NA, _CA * 128, _STAGE)

    cp_pos = pltpu.async_copy(pos_hbm, pos_v, sem_in)
    cp_edge = pltpu.async_copy(
        edge_hbm.at[:, pl.ds(col_base * 128, _STAGE)], ev, sem_in)

    zeros = jnp.zeros((_L,), jnp.float32)

    def zero_body(k, _):
        for j in range(16):
            facc[pl.ds(k * 256 + j * _L, _L)] = zeros
        return 0

    lax.fori_loop(0, _FLAT // 256, zero_body, 0)

    cp_pos.wait()
    cp_edge.wait()

    one = jnp.float32(1.0)
    half = jnp.float32(0.5)
    fzero = jnp.float32(0.0)
    lane = lax.iota(jnp.int32, _L)

    @plsc.parallel_loop(0, _STEPS, 1, unroll=8,
                        carry=jnp.zeros((_L,), jnp.float32))
    def eacc(k, acc):
        s = ev[0, pl.ds(k * _L, _L)]
        t = ev[1, pl.ds(k * _L, _L)]
        valid = (lane + k * _L) < cnt
        s3 = s + s + s
        t3 = t + t + t
        xi = plsc.load_gather(pos_v, [s3])
        yi = plsc.load_gather(pos_v, [s3 + 1])
        zi = plsc.load_gather(pos_v, [s3 + 2])
        xj = plsc.load_gather(pos_v, [t3])
        yj = plsc.load_gather(pos_v, [t3 + 1])
        zj = plsc.load_gather(pos_v, [t3 + 2])
        dx = xj - xi
        dy = yj - yi
        dz = zj - zi
        d2 = dx * dx + dy * dy + dz * dz
        r = _fast_rsqrt(d2)
        d = d2 * r
        mask = d2 < one
        omd = one - d
        e = jnp.where(mask & valid, half * omd * omd, fzero)
        g = jnp.where(mask & valid, one - r, fzero)
        g = jnp.where(d2 == fzero, fzero, g)
        fx = g * dx
        fy = g * dy
        fz = g * dz
        plsc.addupdate_scatter(facc, [s3], fx)
        plsc.addupdate_scatter(facc, [s3 + 1], fy)
        plsc.addupdate_scatter(facc, [s3 + 2], fz)
        plsc.addupdate_scatter(facc, [t3], -fx)
        plsc.addupdate_scatter(facc, [t3 + 1], -fy)
        plsc.addupdate_scatter(facc, [t3 + 2], -fz)
        return acc + e

    e_v[...] = eacc
    pltpu.sync_copy(e_v, eparts_hbm.at[pl.ds(wid * _L, _L)])

    for rnd in range(_NROUND):
        pltpu.sync_copy(facc.at[pl.ds(rnd * _SLAB, _SLAB)],
                        shared.at[pl.ds(sid * _SLAB, _SLAB)])
        plsc.subcore_barrier()

        off = sid * _RED
        cps = [
            pltpu.async_copy(
                shared.at[pl.ds(p * _SLAB + off, _RED)],
                tmp_v.at[pl.ds(p * _RED, _RED)], sem_red)
            for p in range(1, _NS)
        ]
        pltpu.sync_copy(shared.at[pl.ds(off, _RED)], acc_v)
        for cp in cps:
            cp.wait()

        def add_body(k, _):
            acc = acc_v[pl.ds(k * _L, _L)]
            for p in range(1, _NS):
                acc = acc + tmp_v[pl.ds(p * _RED + k * _L, _L)]
            acc_v[pl.ds(k * _L, _L)] = acc
            return 0

        lax.fori_loop(0, _RED // _L, add_body, 0)
        pltpu.sync_copy(
            acc_v,
            fparts_hbm.at[pl.ds(cid * _FLAT + rnd * _SLAB + off, _RED)])
        plsc.subcore_barrier()


_sc_call = pl.kernel(
    _sc_body,
    out_type=(
        jax.ShapeDtypeStruct((_NC * _FLAT,), jnp.float32),
        jax.ShapeDtypeStruct((_NW * _L,), jnp.float32),
    ),
    mesh=plsc.VectorSubcoreMesh(
        core_axis_name="c", subcore_axis_name="s",
        num_cores=_NC, num_subcores=_NS),
    scratch_types=[
        pltpu.VMEM((_W,), jnp.float32),
        pltpu.VMEM((2, _STAGE), jnp.int32),
        pltpu.VMEM((_FLAT,), jnp.float32),
        pltpu.VMEM((_L,), jnp.float32),
        pltpu.VMEM((_NS * _RED,), jnp.float32),
        pltpu.VMEM((_RED,), jnp.float32),
        pltpu.VMEM_SHARED((_NS * _SLAB,), jnp.float32),
        pltpu.SemaphoreType.DMA,
        pltpu.SemaphoreType.DMA,
    ],
    compiler_params=pltpu.CompilerParams(needs_layout_passes=False),
)


def _tc_body(fp_ref, ep_ref, f_ref, e_ref):
    f_ref[...] = fp_ref[0] + fp_ref[1]
    e_ref[0, 0] = jnp.float32(0.5) * jnp.sum(ep_ref[...])


_tc_call = pl.pallas_call(
    _tc_body,
    out_shape=(
        jax.ShapeDtypeStruct((_FLAT // 128, 128), jnp.float32),
        jax.ShapeDtypeStruct((1, 1), jnp.float32),
    ),
    grid=(),
    in_specs=[
        pl.BlockSpec(memory_space=pltpu.VMEM),
        pl.BlockSpec(memory_space=pltpu.VMEM),
    ],
    out_specs=(
        pl.BlockSpec(memory_space=pltpu.VMEM),
        pl.BlockSpec(memory_space=pltpu.SMEM),
    ),
)


@jax.jit
def kernel(positions, edge_index):
    pos_flat = positions.reshape(_W)
    fparts, eparts = _sc_call(pos_flat, edge_index)
    f2d, e = _tc_call(fparts.reshape(_NC, _FLAT // 128, 128),
                      eparts.reshape(4, 128))
    forces = f2d.reshape(_FLAT)[:_W].reshape(_N, 3)
    return e[0, 0], forces


# speedup vs baseline: 106.3231x; 1.0019x over previous
import jax
import jax.numpy as jnp
from jax import lax
from jax.experimental import pallas as pl
from jax.experimental.pallas import tpu as pltpu
from jax.experimental.pallas import tpu_sc as plsc

_N = 10000
_E = 640000
_W = 3 * _N
_FLAT = 30720
_NC = 2
_NS = 16
_L = 16
_NW = _NC * _NS
_COLS = _E // 128
_CA = _COLS // _NW
_NB = _COLS - _CA * _NW
_NA = _NW - _NB
_STAGE = (_CA + 1) * 128
_STEPS = _STAGE // _L
_NROUND = 3
_SLAB = _FLAT // _NROUND
_RED = _SLAB // _NS


def _fast_rsqrt(d2):
    half = jnp.float32(0.5)
    three_half = jnp.float32(1.5)
    ibits = plsc.bitcast(d2, jnp.int32)
    y = plsc.bitcast(jnp.int32(0x5F3759DF) - lax.shift_right_logical(ibits, 1),
                     jnp.float32)
    hd2 = half * d2
    y = y * (three_half - hd2 * y * y)
    y = y * (three_half - hd2 * y * y)
    y = y * (three_half - hd2 * y * y)
    return y


def _sc_body(pos_hbm, edge_hbm, fparts_hbm, eparts_hbm,
             pos_v, ev, facc, e_v, tmp_v, acc_v, shared,
             sem_in, sem_red):
    cid = lax.axis_index("c")
    sid = lax.axis_index("s")
    wid = sid * _NC + cid
    col_base = jnp.where(wid < _NA, wid * _CA, _NA * _CA + (wid - _NA) * (_CA + 1))
    cnt = jnp.where(wid < _NA, _CA * 128, _STAGE)

    cp_pos = pltpu.async_copy(pos_hbm, pos_v, sem_in)
    cp_edge = pltpu.async_copy(
        edge_hbm.at[:, pl.ds(col_base * 128, _STAGE)], ev, sem_in)

    zeros = jnp.zeros((_L,), jnp.float32)

    def zero_body(k, _):
        for j in range(16):
            facc[pl.ds(k * 256 + j * _L, _L)] = zeros
        return 0

    lax.fori_loop(0, _FLAT // 256, zero_body, 0)

    cp_pos.wait()
    cp_edge.wait()

    one = jnp.float32(1.0)
    half = jnp.float32(0.5)
    fzero = jnp.float32(0.0)
    lane = lax.iota(jnp.int32, _L)

    _scope_main = jax.named_scope("edge_loop")
    _scope_main.__enter__()

    @plsc.parallel_loop(0, _STEPS, 1, unroll=8,
                        carry=jnp.zeros((_L,), jnp.float32))
    def eacc(k, acc):
        s = ev[0, pl.ds(k * _L, _L)]
        t = ev[1, pl.ds(k * _L, _L)]
        valid = (lane + k * _L) < cnt
        s3 = s + s + s
        t3 = t + t + t
        xi = plsc.load_gather(pos_v, [s3])
        yi = plsc.load_gather(pos_v, [s3 + 1])
        zi = plsc.load_gather(pos_v, [s3 + 2])
        xj = plsc.load_gather(pos_v, [t3])
        yj = plsc.load_gather(pos_v, [t3 + 1])
        zj = plsc.load_gather(pos_v, [t3 + 2])
        dx = xj - xi
        dy = yj - yi
        dz = zj - zi
        d2 = dx * dx + dy * dy + dz * dz
        r = _fast_rsqrt(d2)
        d = d2 * r
        mask = d2 < one
        omd = one - d
        e = jnp.where(mask & valid, half * omd * omd, fzero)
        g = jnp.where(mask & valid, one - r, fzero)
        g = jnp.where(d2 == fzero, fzero, g)
        fx = g * dx
        fy = g * dy
        fz = g * dz
        plsc.addupdate_scatter(facc, [s3], fx)
        plsc.addupdate_scatter(facc, [s3 + 1], fy)
        plsc.addupdate_scatter(facc, [s3 + 2], fz)
        plsc.addupdate_scatter(facc, [t3], -fx)
        plsc.addupdate_scatter(facc, [t3 + 1], -fy)
        plsc.addupdate_scatter(facc, [t3 + 2], -fz)
        return acc + e

    _scope_main.__exit__(None, None, None)

    e_v[...] = eacc
    pltpu.sync_copy(e_v, eparts_hbm.at[pl.ds(wid * _L, _L)])

    _scope_red = jax.named_scope("reduce")
    _scope_red.__enter__()

    for rnd in range(_NROUND):
        pltpu.sync_copy(facc.at[pl.ds(rnd * _SLAB, _SLAB)],
                        shared.at[pl.ds(sid * _SLAB, _SLAB)])
        plsc.subcore_barrier()

        off = sid * _RED
        cps = [
            pltpu.async_copy(
                shared.at[pl.ds(p * _SLAB + off, _RED)],
                tmp_v.at[pl.ds(p * _RED, _RED)], sem_red)
            for p in range(1, _NS)
        ]
        pltpu.sync_copy(shared.at[pl.ds(off, _RED)], acc_v)
        for cp in cps:
            cp.wait()

        def add_body(k, _):
            acc = acc_v[pl.ds(k * _L, _L)]
            for p in range(1, _NS):
                acc = acc + tmp_v[pl.ds(p * _RED + k * _L, _L)]
            acc_v[pl.ds(k * _L, _L)] = acc
            return 0

        lax.fori_loop(0, _RED // _L, add_body, 0)
        pltpu.sync_copy(
            acc_v,
            fparts_hbm.at[pl.ds(cid * _FLAT + rnd * _SLAB + off, _RED)])
        plsc.subcore_barrier()
    _scope_red.__exit__(None, None, None)


_sc_call = pl.kernel(
    _sc_body,
    out_type=(
        jax.ShapeDtypeStruct((_NC * _FLAT,), jnp.float32),
        jax.ShapeDtypeStruct((_NW * _L,), jnp.float32),
    ),
    mesh=plsc.VectorSubcoreMesh(
        core_axis_name="c", subcore_axis_name="s",
        num_cores=_NC, num_subcores=_NS),
    scratch_types=[
        pltpu.VMEM((_W,), jnp.float32),
        pltpu.VMEM((2, _STAGE), jnp.int32),
        pltpu.VMEM((_FLAT,), jnp.float32),
        pltpu.VMEM((_L,), jnp.float32),
        pltpu.VMEM((_NS * _RED,), jnp.float32),
        pltpu.VMEM((_RED,), jnp.float32),
        pltpu.VMEM_SHARED((_NS * _SLAB,), jnp.float32),
        pltpu.SemaphoreType.DMA,
        pltpu.SemaphoreType.DMA,
    ],
    compiler_params=pltpu.CompilerParams(needs_layout_passes=False),
)


def _tc_body(fp_ref, ep_ref, f_ref, e_ref):
    f_ref[...] = fp_ref[0] + fp_ref[1]
    e_ref[0, 0] = jnp.float32(0.5) * jnp.sum(ep_ref[...])


_tc_call = pl.pallas_call(
    _tc_body,
    out_shape=(
        jax.ShapeDtypeStruct((_FLAT // 128, 128), jnp.float32),
        jax.ShapeDtypeStruct((1, 1), jnp.float32),
    ),
    grid=(),
    in_specs=[
        pl.BlockSpec(memory_space=pltpu.VMEM),
        pl.BlockSpec(memory_space=pltpu.VMEM),
    ],
    out_specs=(
        pl.BlockSpec(memory_space=pltpu.VMEM),
        pl.BlockSpec(memory_space=pltpu.SMEM),
    ),
)


@jax.jit
def kernel(positions, edge_index):
    pos_flat = positions.reshape(_W)
    fparts, eparts = _sc_call(pos_flat, edge_index)
    f2d, e = _tc_call(fparts.reshape(_NC, _FLAT // 128, 128),
                      eparts.reshape(4, 128))
    forces = f2d.reshape(_FLAT)[:_W].reshape(_N, 3)
    return e[0, 0], forces


# speedup vs baseline: 135.4929x; 1.2744x over previous
import jax
import jax.numpy as jnp
from jax import lax
from jax.experimental import pallas as pl
from jax.experimental.pallas import tpu as pltpu
from jax.experimental.pallas import tpu_sc as plsc

_N = 10000
_E = 640000
_W = 3 * _N
_FLAT = 30720
_NC = 2
_NS = 16
_L = 16
_NW = _NC * _NS
_COLS = _E // 128
_CA = _COLS // _NW
_NB = _COLS - _CA * _NW
_NA = _NW - _NB
_STAGE = (_CA + 1) * 128
_STEPS = _STAGE // _L
_NROUND = 3
_SLAB = _FLAT // _NROUND
_RED = _SLAB // _NS


def _fast_rsqrt(d2):
    half = jnp.float32(0.5)
    three_half = jnp.float32(1.5)
    ibits = plsc.bitcast(d2, jnp.int32)
    y = plsc.bitcast(jnp.int32(0x5F3759DF) - lax.shift_right_logical(ibits, 1),
                     jnp.float32)
    hd2 = half * d2
    y = y * (three_half - hd2 * y * y)
    y = y * (three_half - hd2 * y * y)
    return y


def _sc_body(pos_hbm, edge_hbm, fparts_hbm, eparts_hbm,
             pos_v, ev, fax, fay, faz, e_v, tmp_v, acc_v, shared,
             sem_in, sem_red):
    cid = lax.axis_index("c")
    sid = lax.axis_index("s")
    wid = sid * _NC + cid
    col_base = jnp.where(wid < _NA, wid * _CA, _NA * _CA + (wid - _NA) * (_CA + 1))
    cnt = jnp.where(wid < _NA, _CA * 128, _STAGE)

    cp_pos = pltpu.async_copy(pos_hbm, pos_v, sem_in)
    cp_edge = pltpu.async_copy(
        edge_hbm.at[:, pl.ds(col_base * 128, _STAGE)], ev, sem_in)

    zeros = jnp.zeros((_L,), jnp.float32)

    def zero_body(k, _):
        for ref in (fax, fay, faz):
            for j in range(16):
                ref[pl.ds(k * 256 + j * _L, _L)] = zeros
        return 0

    lax.fori_loop(0, _SLAB // 256, zero_body, 0)

    cp_pos.wait()
    cp_edge.wait()

    one = jnp.float32(1.0)
    half = jnp.float32(0.5)
    fzero = jnp.float32(0.0)
    lane = lax.iota(jnp.int32, _L)

    _scope_main = jax.named_scope("edge_loop")
    _scope_main.__enter__()

    @plsc.parallel_loop(0, _STEPS, 1, unroll=8,
                        carry=jnp.zeros((_L,), jnp.float32))
    def eacc(k, acc):
        s = ev[0, pl.ds(k * _L, _L)]
        t = ev[1, pl.ds(k * _L, _L)]
        valid = (lane + k * _L) < cnt
        s3 = s + s + s
        t3 = t + t + t
        xi = plsc.load_gather(pos_v, [s3])
        yi = plsc.load_gather(pos_v, [s3 + 1])
        zi = plsc.load_gather(pos_v, [s3 + 2])
        xj = plsc.load_gather(pos_v, [t3])
        yj = plsc.load_gather(pos_v, [t3 + 1])
        zj = plsc.load_gather(pos_v, [t3 + 2])
        dx = xj - xi
        dy = yj - yi
        dz = zj - zi
        d2 = dx * dx + dy * dy + dz * dz
        r = _fast_rsqrt(d2)
        d = d2 * r
        mask = d2 < one
        omd = one - d
        e = jnp.where(mask & valid, half * omd * omd, fzero)
        g = jnp.where(mask & valid, one - r, fzero)
        g = jnp.where(d2 == fzero, fzero, g)
        fx = g * dx
        fy = g * dy
        fz = g * dz
        plsc.addupdate_scatter(fax, [s], fx)
        plsc.addupdate_scatter(fay, [s], fy)
        plsc.addupdate_scatter(faz, [s], fz)
        plsc.addupdate_scatter(fax, [t], -fx)
        plsc.addupdate_scatter(fay, [t], -fy)
        plsc.addupdate_scatter(faz, [t], -fz)
        return acc + e

    _scope_main.__exit__(None, None, None)

    e_v[...] = eacc
    pltpu.sync_copy(e_v, eparts_hbm.at[pl.ds(wid * _L, _L)])

    _scope_red = jax.named_scope("reduce")
    _scope_red.__enter__()

    for rnd, facc in enumerate((fax, fay, faz)):
        pltpu.sync_copy(facc, shared.at[pl.ds(sid * _SLAB, _SLAB)])
        plsc.subcore_barrier()

        off = sid * _RED
        cps = [
            pltpu.async_copy(
                shared.at[pl.ds(p * _SLAB + off, _RED)],
                tmp_v.at[pl.ds(p * _RED, _RED)], sem_red)
            for p in range(1, _NS)
        ]
        pltpu.sync_copy(shared.at[pl.ds(off, _RED)], acc_v)
        for cp in cps:
            cp.wait()

        def add_body(k, _):
            acc = acc_v[pl.ds(k * _L, _L)]
            for p in range(1, _NS):
                acc = acc + tmp_v[pl.ds(p * _RED + k * _L, _L)]
            acc_v[pl.ds(k * _L, _L)] = acc
            return 0

        lax.fori_loop(0, _RED // _L, add_body, 0)
        pltpu.sync_copy(
            acc_v,
            fparts_hbm.at[pl.ds(cid * _FLAT + rnd * _SLAB + off, _RED)])
        plsc.subcore_barrier()
    _scope_red.__exit__(None, None, None)


_sc_call = pl.kernel(
    _sc_body,
    out_type=(
        jax.ShapeDtypeStruct((_NC * _FLAT,), jnp.float32),
        jax.ShapeDtypeStruct((_NW * _L,), jnp.float32),
    ),
    mesh=plsc.VectorSubcoreMesh(
        core_axis_name="c", subcore_axis_name="s",
        num_cores=_NC, num_subcores=_NS),
    scratch_types=[
        pltpu.VMEM((_W,), jnp.float32),
        pltpu.VMEM((2, _STAGE), jnp.int32),
        pltpu.VMEM((_SLAB,), jnp.float32),
        pltpu.VMEM((_SLAB,), jnp.float32),
        pltpu.VMEM((_SLAB,), jnp.float32),
        pltpu.VMEM((_L,), jnp.float32),
        pltpu.VMEM((_NS * _RED,), jnp.float32),
        pltpu.VMEM((_RED,), jnp.float32),
        pltpu.VMEM_SHARED((_NS * _SLAB,), jnp.float32),
        pltpu.SemaphoreType.DMA,
        pltpu.SemaphoreType.DMA,
    ],
    compiler_params=pltpu.CompilerParams(needs_layout_passes=False),
)


def _tc_body(fp_ref, ep_ref, f_ref, e_ref):
    f_ref[...] = fp_ref[0] + fp_ref[1]
    e_ref[0, 0] = jnp.float32(0.5) * jnp.sum(ep_ref[...])


_tc_call = pl.pallas_call(
    _tc_body,
    out_shape=(
        jax.ShapeDtypeStruct((_FLAT // 128, 128), jnp.float32),
        jax.ShapeDtypeStruct((1, 1), jnp.float32),
    ),
    grid=(),
    in_specs=[
        pl.BlockSpec(memory_space=pltpu.VMEM),
        pl.BlockSpec(memory_space=pltpu.VMEM),
    ],
    out_specs=(
        pl.BlockSpec(memory_space=pltpu.VMEM),
        pl.BlockSpec(memory_space=pltpu.SMEM),
    ),
)


@jax.jit
def kernel(positions, edge_index):
    pos_flat = positions.reshape(_W)
    fparts, eparts = _sc_call(pos_flat, edge_index)
    f2d, e = _tc_call(fparts.reshape(_NC, _FLAT // 128, 128),
                      eparts.reshape(4, 128))
    flat = f2d.reshape(_FLAT)
    forces = jnp.stack(
        [flat[:_N], flat[_SLAB:_SLAB + _N], flat[2 * _SLAB:2 * _SLAB + _N]],
        axis=1)
    return e[0, 0], forces
